# baseline (speedup 1.0000x reference)
import functools

import jax
import jax.numpy as jnp
from jax import lax
from jax.experimental import pallas as pl
from jax.experimental.pallas import tpu as pltpu
from jax.experimental.pallas import tpu_sc as plsc

_B, _S, _D = 4, 4096, 768
_N = _B * _S
_VLEN, _TLEN = 100, 500
_VPAD, _TPAD = 128, 512

_N_SC = 8192
_N_TC = _N - _N_SC

_NC, _NS = 2, 16
_NW = _NC * _NS
_TPW = _N_SC // _NW
_T = 16
_NCHUNK = _TPW // _T
_NBUF = 2
_LANES = 16
_DREGS = _D // _LANES

_mesh = plsc.VectorSubcoreMesh(
    core_axis_name="c", subcore_axis_name="s", num_cores=_NC, num_subcores=_NS
)


@functools.partial(
    pl.kernel,
    out_type=jax.ShapeDtypeStruct((_N, _D), jnp.float32),
    mesh=_mesh,
    scratch_types=[
        pltpu.VMEM((_NCHUNK, _T), jnp.int32),
        pltpu.VMEM((_NCHUNK, _T), jnp.int32),
        pltpu.VMEM((_NBUF, _T, _D), jnp.float32),
        pltpu.VMEM((_NBUF, _T, _D), jnp.float32),
        pltpu.VMEM((_NBUF, _T, _D), jnp.float32),
        pltpu.VMEM((_NBUF, _T, _D), jnp.float32),
        pltpu.SemaphoreType.DMA,
        pltpu.SemaphoreType.DMA,
        pltpu.SemaphoreType.DMA,
        pltpu.SemaphoreType.DMA,
    ],
)
def _sc_embed_add(x_hbm, vs_hbm, ls_hbm, vtab_hbm, ttab_hbm, out_hbm,
                  vidx, tidx, xbuf, vbuf, tbuf, obuf,
                  lsem0, lsem1, wsem0, wsem1):
    wid = lax.axis_index("s") * _NC + lax.axis_index("c")
    base = wid * _TPW
    lsems = (lsem0, lsem1)
    wsems = (wsem0, wsem1)

    pltpu.sync_copy(vs_hbm.at[wid], vidx)
    pltpu.sync_copy(ls_hbm.at[wid], tidx)

    def start_loads(j, b):
        row0 = base + j * _T
        pltpu.async_copy(x_hbm.at[pl.ds(row0, _T)], xbuf.at[b], lsems[b])
        pltpu.async_copy(vtab_hbm.at[vidx.at[j]], vbuf.at[b], lsems[b])
        pltpu.async_copy(ttab_hbm.at[tidx.at[j]], tbuf.at[b], lsems[b])

    def wait_loads(j, b):
        row0 = base + j * _T
        pltpu.make_async_copy(x_hbm.at[pl.ds(row0, _T)], xbuf.at[b], lsems[b]).wait()
        pltpu.make_async_copy(vtab_hbm.at[vidx.at[j]], vbuf.at[b], lsems[b]).wait()
        pltpu.make_async_copy(ttab_hbm.at[tidx.at[j]], tbuf.at[b], lsems[b]).wait()

    def start_wb(j, b):
        row0 = base + j * _T
        pltpu.async_copy(obuf.at[b], out_hbm.at[pl.ds(row0, _T)], wsems[b])

    def wait_wb(j, b):
        row0 = base + j * _T
        pltpu.make_async_copy(obuf.at[b], out_hbm.at[pl.ds(row0, _T)], wsems[b]).wait()

    def compute(b):
        def body(t, carry):
            for d in range(_DREGS):
                sl = pl.ds(d * _LANES, _LANES)
                obuf[b, t, sl] = xbuf[b, t, sl] + vbuf[b, t, sl] + tbuf[b, t, sl]
            return carry
        lax.fori_loop(0, _T, body, 0)

    start_loads(0, 0)
    start_loads(1, 1)

    def group(g, carry):
        for b in range(_NBUF):
            j = g * _NBUF + b
            wait_loads(j, b)
            @pl.when(g > 0)
            def _():
                wait_wb(j - _NBUF, b)
            compute(b)
            start_wb(j, b)
            @pl.when(g < _NCHUNK // _NBUF - 1)
            def _():
                start_loads(j + _NBUF, b)
        return carry

    lax.fori_loop(0, _NCHUNK // _NBUF, group, 0)

    wait_wb(_NCHUNK - 2, 0)
    wait_wb(_NCHUNK - 1, 1)


_TB = 512
_NB_SC = _N_SC // _TB
_NB_TC = _N_TC // _TB


def _tc_body(x_ref, vs_ref, ls_ref, vt_ref, tt_ref, o_ref):
    vs = vs_ref[0, 0, :]
    ls = ls_ref[0, 0, :]
    ohv = (vs[:, None] == lax.broadcasted_iota(jnp.int32, (_TB, _VPAD), 1))
    oht = (ls[:, None] == lax.broadcasted_iota(jnp.int32, (_TB, _TPAD), 1))
    ve = jnp.dot(ohv.astype(jnp.bfloat16), vt_ref[...],
                 preferred_element_type=jnp.float32)
    te = jnp.dot(oht.astype(jnp.bfloat16), tt_ref[...],
                 preferred_element_type=jnp.float32)
    o_ref[...] = x_ref[...] + ve + te


_tc_embed_add = pl.pallas_call(
    _tc_body,
    grid=(_NB_TC,),
    in_specs=[
        pl.BlockSpec((_TB, _D), lambda i: (i, 0)),
        pl.BlockSpec((1, 1, _TB), lambda i: (i, 0, 0)),
        pl.BlockSpec((1, 1, _TB), lambda i: (i, 0, 0)),
        pl.BlockSpec((_VPAD, _D), lambda i: (0, 0)),
        pl.BlockSpec((_TPAD, _D), lambda i: (0, 0)),
    ],
    out_specs=pl.BlockSpec((_TB, _D), lambda i: (i, 0)),
    out_shape=jax.ShapeDtypeStruct((_N_TC, _D), jnp.float32),
)


def kernel(x, variable_seq, lead_time_seq, var_table, time_table):
    xf = x.reshape(_N, _D)
    vs_flat = variable_seq.reshape(_N).astype(jnp.int32)
    ls_flat = lead_time_seq.reshape(_N).astype(jnp.int32)

    vs_sc = vs_flat[:_N_SC].reshape(_NW, _NCHUNK, _T)
    ls_sc = ls_flat[:_N_SC].reshape(_NW, _NCHUNK, _T)
    out_sc = _sc_embed_add(xf, vs_sc, ls_sc, var_table, time_table)

    x_tail = xf[_N_SC:]
    vs3 = vs_flat[_N_SC:].reshape(_NB_TC, 1, _TB)
    ls3 = ls_flat[_N_SC:].reshape(_NB_TC, 1, _TB)
    vtab_bf = jnp.pad(var_table, ((0, _VPAD - _VLEN), (0, 0))).astype(jnp.bfloat16)
    ttab_bf = jnp.pad(time_table, ((0, _TPAD - _TLEN), (0, 0))).astype(jnp.bfloat16)
    out_tc = _tc_embed_add(x_tail, vs3, ls3, vtab_bf, ttab_bf)

    out = lax.dynamic_update_slice(out_sc, out_tc, (_N_SC, 0))
    return out.reshape(_B, _S, _D)


# speedup vs baseline: 1.0261x; 1.0261x over previous
import functools

import jax
import jax.numpy as jnp
from jax import lax
from jax.experimental import pallas as pl
from jax.experimental.pallas import tpu as pltpu
from jax.experimental.pallas import tpu_sc as plsc

_B, _S, _D = 4, 4096, 768
_N = _B * _S
_VLEN, _TLEN = 100, 500
_VPAD, _TPAD = 128, 512

_N_SC = 8192
_N_TC = _N - _N_SC

_NC, _NS = 2, 16
_NW = _NC * _NS
_TPW = _N_SC // _NW
_T = 16
_NCHUNK = _TPW // _T
_NBUF = 2
_LANES = 16
_DREGS = _D // _LANES

_mesh = plsc.VectorSubcoreMesh(
    core_axis_name="c", subcore_axis_name="s", num_cores=_NC, num_subcores=_NS
)


@functools.partial(
    pl.kernel,
    out_type=jax.ShapeDtypeStruct((_N, _D), jnp.float32),
    mesh=_mesh,
    compiler_params=pltpu.CompilerParams(needs_layout_passes=False),
    scratch_types=[
        pltpu.VMEM((_NCHUNK, _T), jnp.int32),
        pltpu.VMEM((_NCHUNK, _T), jnp.int32),
        pltpu.VMEM((_NBUF, _T, _D), jnp.float32),
        pltpu.VMEM((_NBUF, _T, _D // 2), jnp.int32),
        pltpu.VMEM((_NBUF, _T, _D // 2), jnp.int32),
        pltpu.VMEM((_NBUF, _T, _D), jnp.float32),
        pltpu.SemaphoreType.DMA,
        pltpu.SemaphoreType.DMA,
        pltpu.SemaphoreType.DMA,
        pltpu.SemaphoreType.DMA,
    ],
)
def _sc_embed_add(x_hbm, vs_hbm, ls_hbm, vtab_hbm, ttab_hbm, out_hbm,
                  vidx, tidx, xbuf, vbuf, tbuf, obuf,
                  lsem0, lsem1, wsem0, wsem1):
    wid = lax.axis_index("s") * _NC + lax.axis_index("c")
    base = wid * _TPW
    lsems = (lsem0, lsem1)
    wsems = (wsem0, wsem1)

    pltpu.sync_copy(vs_hbm.at[wid], vidx)
    pltpu.sync_copy(ls_hbm.at[wid], tidx)

    def start_loads(j, b):
        row0 = base + j * _T
        pltpu.async_copy(x_hbm.at[pl.ds(row0, _T)], xbuf.at[b], lsems[b])
        pltpu.async_copy(vtab_hbm.at[vidx.at[j]], vbuf.at[b], lsems[b])
        pltpu.async_copy(ttab_hbm.at[tidx.at[j]], tbuf.at[b], lsems[b])

    def wait_loads(j, b):
        row0 = base + j * _T
        pltpu.make_async_copy(x_hbm.at[pl.ds(row0, _T)], xbuf.at[b], lsems[b]).wait()
        pltpu.make_async_copy(vtab_hbm.at[vidx.at[j]], vbuf.at[b], lsems[b]).wait()
        pltpu.make_async_copy(ttab_hbm.at[tidx.at[j]], tbuf.at[b], lsems[b]).wait()

    def start_wb(j, b):
        row0 = base + j * _T
        pltpu.async_copy(obuf.at[b], out_hbm.at[pl.ds(row0, _T)], wsems[b])

    def wait_wb(j, b):
        row0 = base + j * _T
        pltpu.make_async_copy(obuf.at[b], out_hbm.at[pl.ds(row0, _T)], wsems[b]).wait()

    def compute(b):
        def body(t, carry):
            for k in range(_D // 32):
                slw = pl.ds(k * _LANES, _LANES)
                v32 = plsc.bitcast(vbuf[b, t, slw], jnp.bfloat16)
                t32 = plsc.bitcast(tbuf[b, t, slw], jnp.bfloat16)
                va, vb2 = plsc.unpack(v32, format=plsc.PackFormat.INTERLEAVED,
                                      preferred_element_type=jnp.float32)
                ta, tb2 = plsc.unpack(t32, format=plsc.PackFormat.INTERLEAVED,
                                      preferred_element_type=jnp.float32)
                slo = pl.ds(k * 32, _LANES)
                shi = pl.ds(k * 32 + _LANES, _LANES)
                obuf[b, t, slo] = xbuf[b, t, slo] + va + ta
                obuf[b, t, shi] = xbuf[b, t, shi] + vb2 + tb2
            return carry
        lax.fori_loop(0, _T, body, 0)

    start_loads(0, 0)
    start_loads(1, 1)

    def group(g, carry):
        for b in range(_NBUF):
            j = g * _NBUF + b
            wait_loads(j, b)
            @pl.when(g > 0)
            def _():
                wait_wb(j - _NBUF, b)
            compute(b)
            start_wb(j, b)
            @pl.when(g < _NCHUNK // _NBUF - 1)
            def _():
                start_loads(j + _NBUF, b)
        return carry

    lax.fori_loop(0, _NCHUNK // _NBUF, group, 0)

    wait_wb(_NCHUNK - 2, 0)
    wait_wb(_NCHUNK - 1, 1)


_TB = 512
_NB_SC = _N_SC // _TB
_NB_TC = _N_TC // _TB


def _tc_body(prev_ref, x_ref, vs_ref, ls_ref, vt_ref, tt_ref, o_ref):
    del prev_ref
    vs = vs_ref[0, 0, :]
    ls = ls_ref[0, 0, :]
    ohv = (vs[:, None] == lax.broadcasted_iota(jnp.int32, (_TB, _VPAD), 1))
    oht = (ls[:, None] == lax.broadcasted_iota(jnp.int32, (_TB, _TPAD), 1))
    ve = jnp.dot(ohv.astype(jnp.bfloat16), vt_ref[...],
                 preferred_element_type=jnp.float32)
    te = jnp.dot(oht.astype(jnp.bfloat16), tt_ref[...],
                 preferred_element_type=jnp.float32)
    o_ref[...] = x_ref[...] + ve + te


_tc_embed_add = pl.pallas_call(
    _tc_body,
    grid=(_NB_TC,),
    in_specs=[
        pl.BlockSpec(memory_space=pl.ANY),
        pl.BlockSpec((_TB, _D), lambda i: (_NB_SC + i, 0)),
        pl.BlockSpec((1, 1, _TB), lambda i: (_NB_SC + i, 0, 0)),
        pl.BlockSpec((1, 1, _TB), lambda i: (_NB_SC + i, 0, 0)),
        pl.BlockSpec((_VPAD, _D), lambda i: (0, 0)),
        pl.BlockSpec((_TPAD, _D), lambda i: (0, 0)),
    ],
    out_specs=pl.BlockSpec((_TB, _D), lambda i: (_NB_SC + i, 0)),
    out_shape=jax.ShapeDtypeStruct((_N, _D), jnp.float32),
    input_output_aliases={0: 0},
)


def _perm_bf16(tab):
    r = tab.shape[0]
    pb = (tab.reshape(r, _D // 32, 2, 16).transpose(0, 1, 3, 2)
          .reshape(r, _D).astype(jnp.bfloat16))
    return jax.lax.bitcast_convert_type(pb.reshape(r, _D // 2, 2), jnp.int32)


def kernel(x, variable_seq, lead_time_seq, var_table, time_table):
    xf = x.reshape(_N, _D)
    vs_flat = variable_seq.reshape(_N).astype(jnp.int32)
    ls_flat = lead_time_seq.reshape(_N).astype(jnp.int32)

    vs_sc = vs_flat[:_N_SC].reshape(_NW, _NCHUNK, _T)
    ls_sc = ls_flat[:_N_SC].reshape(_NW, _NCHUNK, _T)
    out_sc = _sc_embed_add(xf, vs_sc, ls_sc, _perm_bf16(var_table),
                           _perm_bf16(time_table))

    vs3 = vs_flat.reshape(_N // _TB, 1, _TB)
    ls3 = ls_flat.reshape(_N // _TB, 1, _TB)
    vtab_bf = jnp.pad(var_table, ((0, _VPAD - _VLEN), (0, 0))).astype(jnp.bfloat16)
    ttab_bf = jnp.pad(time_table, ((0, _TPAD - _TLEN), (0, 0))).astype(jnp.bfloat16)
    out = _tc_embed_add(out_sc, xf, vs3, ls3, vtab_bf, ttab_bf)

    return out.reshape(_B, _S, _D)


# speedup vs baseline: 1.2065x; 1.1758x over previous
import functools

import jax
import jax.numpy as jnp
from jax import lax
from jax.experimental import pallas as pl
from jax.experimental.pallas import tpu as pltpu
from jax.experimental.pallas import tpu_sc as plsc

_B, _S, _D = 4, 4096, 768
_N = _B * _S
_VLEN, _TLEN = 100, 500
_VPAD, _TPAD = 128, 512

_N_SC = 6144
_N_TC = _N - _N_SC

_NC, _NS = 2, 16
_NW = _NC * _NS
_TPW = _N_SC // _NW
_T = 16
_NCHUNK = _TPW // _T
_NBUF = 2
_LANES = 16
_DREGS = _D // _LANES

_mesh = plsc.VectorSubcoreMesh(
    core_axis_name="c", subcore_axis_name="s", num_cores=_NC, num_subcores=_NS
)


@functools.partial(
    pl.kernel,
    out_type=jax.ShapeDtypeStruct((_N, _D), jnp.float32),
    mesh=_mesh,
    scratch_types=[
        pltpu.VMEM((_NCHUNK, _T), jnp.int32),
        pltpu.VMEM((_NCHUNK, _T), jnp.int32),
        pltpu.VMEM((_NBUF, _T, _D), jnp.float32),
        pltpu.VMEM((_NBUF, _T, _D), jnp.float32),
        pltpu.VMEM((_NBUF, _T, _D), jnp.float32),
        pltpu.VMEM((_NBUF, _T, _D), jnp.float32),
        pltpu.SemaphoreType.DMA,
        pltpu.SemaphoreType.DMA,
        pltpu.SemaphoreType.DMA,
        pltpu.SemaphoreType.DMA,
    ],
)
def _sc_embed_add(x_hbm, vs_hbm, ls_hbm, vtab_hbm, ttab_hbm, out_hbm,
                  vidx, tidx, xbuf, vbuf, tbuf, obuf,
                  lsem0, lsem1, wsem0, wsem1):
    wid = lax.axis_index("s") * _NC + lax.axis_index("c")
    base = wid * _TPW
    lsems = (lsem0, lsem1)
    wsems = (wsem0, wsem1)

    pltpu.sync_copy(vs_hbm.at[wid], vidx)
    pltpu.sync_copy(ls_hbm.at[wid], tidx)

    def start_loads(j, b):
        row0 = base + j * _T
        pltpu.async_copy(x_hbm.at[pl.ds(row0, _T)], xbuf.at[b], lsems[b])
        pltpu.async_copy(vtab_hbm.at[vidx.at[j]], vbuf.at[b], lsems[b])
        pltpu.async_copy(ttab_hbm.at[tidx.at[j]], tbuf.at[b], lsems[b])

    def wait_loads(j, b):
        row0 = base + j * _T
        pltpu.make_async_copy(x_hbm.at[pl.ds(row0, _T)], xbuf.at[b], lsems[b]).wait()
        pltpu.make_async_copy(vtab_hbm.at[vidx.at[j]], vbuf.at[b], lsems[b]).wait()
        pltpu.make_async_copy(ttab_hbm.at[tidx.at[j]], tbuf.at[b], lsems[b]).wait()

    def start_wb(j, b):
        row0 = base + j * _T
        pltpu.async_copy(obuf.at[b], out_hbm.at[pl.ds(row0, _T)], wsems[b])

    def wait_wb(j, b):
        row0 = base + j * _T
        pltpu.make_async_copy(obuf.at[b], out_hbm.at[pl.ds(row0, _T)], wsems[b]).wait()

    def compute(b):
        def body(t, carry):
            for d in range(_DREGS):
                sl = pl.ds(d * _LANES, _LANES)
                obuf[b, t, sl] = xbuf[b, t, sl] + vbuf[b, t, sl] + tbuf[b, t, sl]
            return carry
        lax.fori_loop(0, _T, body, 0)

    start_loads(0, 0)
    start_loads(1, 1)

    def group(g, carry):
        for b in range(_NBUF):
            j = g * _NBUF + b
            wait_loads(j, b)
            @pl.when(g > 0)
            def _():
                wait_wb(j - _NBUF, b)
            compute(b)
            start_wb(j, b)
            @pl.when(g < _NCHUNK // _NBUF - 1)
            def _():
                start_loads(j + _NBUF, b)
        return carry

    lax.fori_loop(0, _NCHUNK // _NBUF, group, 0)

    wait_wb(_NCHUNK - 2, 0)
    wait_wb(_NCHUNK - 1, 1)


_TB = 512
_NB_SC = _N_SC // _TB
_NB_TC = _N_TC // _TB


def _tc_body(prev_ref, x_ref, vs_ref, ls_ref, vt_ref, tt_ref, o_ref):
    del prev_ref
    vs = vs_ref[0, 0, :]
    ls = ls_ref[0, 0, :]
    ohv = (vs[:, None] == lax.broadcasted_iota(jnp.int32, (_TB, _VPAD), 1))
    oht = (ls[:, None] == lax.broadcasted_iota(jnp.int32, (_TB, _TPAD), 1))
    ve = jnp.dot(ohv.astype(jnp.bfloat16), vt_ref[...],
                 preferred_element_type=jnp.float32)
    te = jnp.dot(oht.astype(jnp.bfloat16), tt_ref[...],
                 preferred_element_type=jnp.float32)
    o_ref[...] = x_ref[...] + ve + te


_tc_embed_add = pl.pallas_call(
    _tc_body,
    grid=(_NB_TC,),
    in_specs=[
        pl.BlockSpec(memory_space=pl.ANY),
        pl.BlockSpec((_TB, _D), lambda i: (_NB_SC + i, 0)),
        pl.BlockSpec((1, 1, _TB), lambda i: (_NB_SC + i, 0, 0)),
        pl.BlockSpec((1, 1, _TB), lambda i: (_NB_SC + i, 0, 0)),
        pl.BlockSpec((_VPAD, _D), lambda i: (0, 0)),
        pl.BlockSpec((_TPAD, _D), lambda i: (0, 0)),
    ],
    out_specs=pl.BlockSpec((_TB, _D), lambda i: (_NB_SC + i, 0)),
    out_shape=jax.ShapeDtypeStruct((_N, _D), jnp.float32),
    input_output_aliases={0: 0},
)


def kernel(x, variable_seq, lead_time_seq, var_table, time_table):
    xf = x.reshape(_N, _D)
    vs_flat = variable_seq.reshape(_N).astype(jnp.int32)
    ls_flat = lead_time_seq.reshape(_N).astype(jnp.int32)

    vs_sc = vs_flat[:_N_SC].reshape(_NW, _NCHUNK, _T)
    ls_sc = ls_flat[:_N_SC].reshape(_NW, _NCHUNK, _T)
    out_sc = _sc_embed_add(xf, vs_sc, ls_sc, var_table, time_table)

    vs3 = vs_flat.reshape(_N // _TB, 1, _TB)
    ls3 = ls_flat.reshape(_N // _TB, 1, _TB)
    vtab_bf = jnp.pad(var_table, ((0, _VPAD - _VLEN), (0, 0))).astype(jnp.bfloat16)
    ttab_bf = jnp.pad(time_table, ((0, _TPAD - _TLEN), (0, 0))).astype(jnp.bfloat16)
    out = _tc_embed_add(out_sc, xf, vs3, ls3, vtab_bf, ttab_bf)

    return out.reshape(_B, _S, _D)


# speedup vs baseline: 1.4449x; 1.1976x over previous
import functools

import jax
import jax.numpy as jnp
from jax import lax
from jax.experimental import pallas as pl
from jax.experimental.pallas import tpu as pltpu
from jax.experimental.pallas import tpu_sc as plsc

_B, _S, _D = 4, 4096, 768
_N = _B * _S
_VLEN, _TLEN = 100, 500
_VPAD, _TPAD = 128, 512

_N_SC = 4096
_N_TC = _N - _N_SC

_NC, _NS = 2, 16
_NW = _NC * _NS
_TPW = _N_SC // _NW
_T = 16
_NCHUNK = _TPW // _T
_NBUF = 2
_LANES = 16
_DREGS = _D // _LANES

_mesh = plsc.VectorSubcoreMesh(
    core_axis_name="c", subcore_axis_name="s", num_cores=_NC, num_subcores=_NS
)


@functools.partial(
    pl.kernel,
    out_type=jax.ShapeDtypeStruct((_N, _D), jnp.float32),
    mesh=_mesh,
    scratch_types=[
        pltpu.VMEM((_NCHUNK, _T), jnp.int32),
        pltpu.VMEM((_NCHUNK, _T), jnp.int32),
        pltpu.VMEM((_NBUF, _T, _D), jnp.float32),
        pltpu.VMEM((_NBUF, _T, _D), jnp.float32),
        pltpu.VMEM((_NBUF, _T, _D), jnp.float32),
        pltpu.VMEM((_NBUF, _T, _D), jnp.float32),
        pltpu.SemaphoreType.DMA,
        pltpu.SemaphoreType.DMA,
        pltpu.SemaphoreType.DMA,
        pltpu.SemaphoreType.DMA,
    ],
)
def _sc_embed_add(x_hbm, vs_hbm, ls_hbm, vtab_hbm, ttab_hbm, out_hbm,
                  vidx, tidx, xbuf, vbuf, tbuf, obuf,
                  lsem0, lsem1, wsem0, wsem1):
    wid = lax.axis_index("s") * _NC + lax.axis_index("c")
    base = wid * _TPW
    lsems = (lsem0, lsem1)
    wsems = (wsem0, wsem1)

    pltpu.sync_copy(vs_hbm.at[wid], vidx)
    pltpu.sync_copy(ls_hbm.at[wid], tidx)

    def start_loads(j, b):
        row0 = base + j * _T
        pltpu.async_copy(x_hbm.at[pl.ds(row0, _T)], xbuf.at[b], lsems[b])
        pltpu.async_copy(vtab_hbm.at[vidx.at[j]], vbuf.at[b], lsems[b])
        pltpu.async_copy(ttab_hbm.at[tidx.at[j]], tbuf.at[b], lsems[b])

    def wait_loads(j, b):
        row0 = base + j * _T
        pltpu.make_async_copy(x_hbm.at[pl.ds(row0, _T)], xbuf.at[b], lsems[b]).wait()
        pltpu.make_async_copy(vtab_hbm.at[vidx.at[j]], vbuf.at[b], lsems[b]).wait()
        pltpu.make_async_copy(ttab_hbm.at[tidx.at[j]], tbuf.at[b], lsems[b]).wait()

    def start_wb(j, b):
        row0 = base + j * _T
        pltpu.async_copy(obuf.at[b], out_hbm.at[pl.ds(row0, _T)], wsems[b])

    def wait_wb(j, b):
        row0 = base + j * _T
        pltpu.make_async_copy(obuf.at[b], out_hbm.at[pl.ds(row0, _T)], wsems[b]).wait()

    def compute(b):
        def body(t, carry):
            for d in range(_DREGS):
                sl = pl.ds(d * _LANES, _LANES)
                obuf[b, t, sl] = xbuf[b, t, sl] + vbuf[b, t, sl] + tbuf[b, t, sl]
            return carry
        lax.fori_loop(0, _T, body, 0)

    start_loads(0, 0)
    start_loads(1, 1)

    def group(g, carry):
        for b in range(_NBUF):
            j = g * _NBUF + b
            wait_loads(j, b)
            @pl.when(g > 0)
            def _():
                wait_wb(j - _NBUF, b)
            compute(b)
            start_wb(j, b)
            @pl.when(g < _NCHUNK // _NBUF - 1)
            def _():
                start_loads(j + _NBUF, b)
        return carry

    lax.fori_loop(0, _NCHUNK // _NBUF, group, 0)

    wait_wb(_NCHUNK - 2, 0)
    wait_wb(_NCHUNK - 1, 1)


_TB = 1024
_NB_SC = _N_SC // _TB
_NB_TC = _N_TC // _TB


def _tc_body(prev_ref, x_ref, vs_ref, ls_ref, vt_ref, tt_ref, o_ref):
    del prev_ref
    vs = vs_ref[0, 0, :]
    ls = ls_ref[0, 0, :]
    ohv = (vs[:, None] == lax.broadcasted_iota(jnp.int32, (_TB, _VPAD), 1))
    oht = (ls[:, None] == lax.broadcasted_iota(jnp.int32, (_TB, _TPAD), 1))
    ve = jnp.dot(ohv.astype(jnp.bfloat16), vt_ref[...],
                 preferred_element_type=jnp.float32)
    te = jnp.dot(oht.astype(jnp.bfloat16), tt_ref[...],
                 preferred_element_type=jnp.float32)
    o_ref[...] = x_ref[...] + ve + te


_tc_embed_add = pl.pallas_call(
    _tc_body,
    grid=(_NB_TC,),
    in_specs=[
        pl.BlockSpec(memory_space=pl.ANY),
        pl.BlockSpec((_TB, _D), lambda i: (_NB_SC + i, 0)),
        pl.BlockSpec((1, 1, _TB), lambda i: (_NB_SC + i, 0, 0)),
        pl.BlockSpec((1, 1, _TB), lambda i: (_NB_SC + i, 0, 0)),
        pl.BlockSpec((_VPAD, _D), lambda i: (0, 0)),
        pl.BlockSpec((_TPAD, _D), lambda i: (0, 0)),
    ],
    out_specs=pl.BlockSpec((_TB, _D), lambda i: (_NB_SC + i, 0)),
    out_shape=jax.ShapeDtypeStruct((_N, _D), jnp.float32),
    input_output_aliases={0: 0},
)


def kernel(x, variable_seq, lead_time_seq, var_table, time_table):
    xf = x.reshape(_N, _D)
    vs_flat = variable_seq.reshape(_N).astype(jnp.int32)
    ls_flat = lead_time_seq.reshape(_N).astype(jnp.int32)

    vs_sc = vs_flat[:_N_SC].reshape(_NW, _NCHUNK, _T)
    ls_sc = ls_flat[:_N_SC].reshape(_NW, _NCHUNK, _T)
    out_sc = _sc_embed_add(xf, vs_sc, ls_sc, var_table, time_table)

    vs3 = vs_flat.reshape(_N // _TB, 1, _TB)
    ls3 = ls_flat.reshape(_N // _TB, 1, _TB)
    vtab_bf = jnp.pad(var_table, ((0, _VPAD - _VLEN), (0, 0))).astype(jnp.bfloat16)
    ttab_bf = jnp.pad(time_table, ((0, _TPAD - _TLEN), (0, 0))).astype(jnp.bfloat16)
    out = _tc_embed_add(out_sc, xf, vs3, ls3, vtab_bf, ttab_bf)

    return out.reshape(_B, _S, _D)
